# Initial kernel scaffold; baseline (speedup 1.0000x reference)
import jax
import jax.numpy as jnp
from jax.experimental import pallas as pl


def kernel(query, support, q_l_conn, q_l_deg, q_r_conn, q_r_deg, s_l_conn, s_l_deg, s_r_conn, s_r_deg, table, gcn_wW, gcn_wb, gcn_b, p1W, p1b, p2W, p2b, ln_g, ln_b, Wih, Whh, bih, bhh):
    raise NotImplementedError("write your pallas kernel here")


# speedup vs baseline: 2.7090x; 2.7090x over previous
import functools

import jax
import jax.numpy as jnp
from jax.experimental import pallas as pl
from jax.experimental.pallas import tpu as pltpu

NB = 50
K = 32
D = 128
DM = 256
RBLK = 128


def _encoder_block_kernel(ent_ref, rel_ref, cen_ref, w_ref, b_ref, out_ref):
    cen = cen_ref[...]
    R = cen.shape[0]

    d_cols, nn_cols = [], []
    for j in range(NB):
        e = ent_ref[:, j, :]
        d_cols.append(jnp.sum(cen * e, axis=-1, keepdims=True))
        nn_cols.append(jnp.sum(e * e, axis=-1, keepdims=True))
    d = jnp.concatenate(d_cols, axis=1)
    nn = jnp.concatenate(nn_cols, axis=1)
    cn = jnp.sum(cen * cen, axis=-1)
    an = jnp.maximum(jnp.sqrt(cn), 1e-8)[:, None]
    bn = jnp.maximum(jnp.sqrt(nn), 1e-8)
    sim = d / (an * bn)

    i_iota = jax.lax.broadcasted_iota(jnp.int32, (R, NB), 1)
    rank = jnp.zeros((R, NB), jnp.float32)
    for j in range(NB):
        sj = sim[:, j:j + 1]
        rank = rank + (sj > sim).astype(jnp.float32)
        rank = rank + ((sj == sim) & (j < i_iota)).astype(jnp.float32)
    w = jnp.where(rank < K, 1.0 / K, 0.0)

    m_rel = jnp.zeros((R, D), jnp.float32)
    m_ent = jnp.zeros((R, D), jnp.float32)
    for j in range(NB):
        wj = w[:, j:j + 1]
        m_rel = m_rel + wj * rel_ref[:, j, :]
        m_ent = m_ent + wj * ent_ref[:, j, :]
    mc = jnp.concatenate([m_rel, m_ent], axis=-1)
    pre = jnp.dot(mc, w_ref[...].T,
                  preferred_element_type=jnp.float32) + b_ref[...]
    out_ref[...] = jnp.tanh(pre)


def _head_kernel(neigh_ref, p1W_ref, p1b_ref, p2W_ref, p2b_ref,
                 ln_g_ref, ln_b_ref, Wih_ref, Whh_ref, bih_ref, bhh_ref,
                 out_ref, B):
    neigh = neigh_ref[...]
    qn = jnp.concatenate([neigh[0:B], neigh[B:2 * B]], axis=1)
    sn = jnp.concatenate([neigh[2 * B:2 * B + 5],
                          neigh[2 * B + 8:2 * B + 13]], axis=1)

    p1W = p1W_ref[...]
    p2W = p2W_ref[...]
    ln_g = ln_g_ref[...]
    ln_b = ln_b_ref[...]

    def enc(x):
        out = jax.nn.relu(jnp.dot(x, p1W.T, preferred_element_type=jnp.float32)
                          + p1b_ref[...])
        out = jnp.dot(out, p2W.T, preferred_element_type=jnp.float32) + p2b_ref[...]
        out = out + x
        m = jnp.mean(out, axis=-1, keepdims=True)
        v = jnp.mean((out - m) ** 2, axis=-1, keepdims=True)
        return (out - m) / jnp.sqrt(v + 1e-5) * ln_g + ln_b

    support_g = jnp.mean(enc(sn), axis=0, keepdims=True)
    query_g = enc(qn)

    Wih = Wih_ref[...]
    Whh = Whh_ref[...]
    bih = bih_ref[...]
    bhh = bhh_ref[...]
    sup_b = jnp.broadcast_to(support_g, (B, DM))

    h_r = jnp.zeros((B, 2 * DM), jnp.float32)
    c = jnp.zeros((B, 2 * DM), jnp.float32)
    h = query_g
    for _ in range(2):
        gates = (jnp.dot(query_g, Wih.T, preferred_element_type=jnp.float32)
                 + bih
                 + jnp.dot(h_r, Whh.T, preferred_element_type=jnp.float32)
                 + bhh)
        i_g = gates[:, 0:2 * DM]
        f_g = gates[:, 2 * DM:4 * DM]
        g_g = gates[:, 4 * DM:6 * DM]
        o_g = gates[:, 6 * DM:8 * DM]
        c = jax.nn.sigmoid(f_g) * c + jax.nn.sigmoid(i_g) * jnp.tanh(g_g)
        h_new = jax.nn.sigmoid(o_g) * jnp.tanh(c)
        h = query_g + h_new[:, :DM]
        h_r = jnp.concatenate([h, sup_b], axis=1)

    qf = h / jnp.maximum(jnp.linalg.norm(h, axis=-1, keepdims=True), 1e-12)
    sv = support_g[0]
    sv = sv / jnp.maximum(jnp.linalg.norm(sv), 1e-12)
    out_ref[...] = jnp.dot(qf, sv[:, None],
                           preferred_element_type=jnp.float32)[:, 0]


def kernel(query, support, q_l_conn, q_l_deg, q_r_conn, q_r_deg,
           s_l_conn, s_l_deg, s_r_conn, s_r_deg, table,
           gcn_wW, gcn_wb, gcn_b, p1W, p1b, p2W, p2b, ln_g, ln_b,
           Wih, Whh, bih, bhh):
    B = query.shape[0]
    FEW = support.shape[0]

    n_rows = 2 * B + 16
    rows_pad = ((n_rows + RBLK - 1) // RBLK) * RBLK
    ids = jnp.zeros((rows_pad,), jnp.int32)
    ids = ids.at[0:B].set(query[:, 0].astype(jnp.int32))
    ids = ids.at[B:2 * B].set(query[:, 1].astype(jnp.int32))
    ids = ids.at[2 * B:2 * B + FEW].set(support[:, 0].astype(jnp.int32))
    ids = ids.at[2 * B + 8:2 * B + 8 + FEW].set(support[:, 1].astype(jnp.int32))
    conn = jnp.zeros((rows_pad, NB, 2), jnp.int32)
    conn = conn.at[0:B].set(q_l_conn.astype(jnp.int32))
    conn = conn.at[B:2 * B].set(q_r_conn.astype(jnp.int32))
    conn = conn.at[2 * B:2 * B + FEW].set(s_l_conn.astype(jnp.int32))
    conn = conn.at[2 * B + 8:2 * B + 8 + FEW].set(s_r_conn.astype(jnp.int32))

    ent_g = jnp.take(table, conn[:, :, 1], axis=0)
    rel_g = jnp.take(table, conn[:, :, 0], axis=0)
    cen_g = jnp.take(table, ids, axis=0)

    nblk = rows_pad // RBLK
    neigh = pl.pallas_call(
        _encoder_block_kernel,
        grid=(nblk,),
        in_specs=[
            pl.BlockSpec((RBLK, NB, D), lambda i: (i, 0, 0)),
            pl.BlockSpec((RBLK, NB, D), lambda i: (i, 0, 0)),
            pl.BlockSpec((RBLK, D), lambda i: (i, 0)),
            pl.BlockSpec((D, 2 * D), lambda i: (0, 0)),
            pl.BlockSpec((D,), lambda i: (0,)),
        ],
        out_specs=pl.BlockSpec((RBLK, D), lambda i: (i, 0)),
        out_shape=jax.ShapeDtypeStruct((rows_pad, D), jnp.float32),
    )(ent_g, rel_g, cen_g, gcn_wW, gcn_wb + gcn_b)

    scores = pl.pallas_call(
        functools.partial(_head_kernel, B=B),
        out_shape=jax.ShapeDtypeStruct((B,), jnp.float32),
        compiler_params=pltpu.CompilerParams(
            vmem_limit_bytes=63 * 1024 * 1024),
    )(neigh, p1W, p1b, p2W, p2b, ln_g, ln_b, Wih, Whh, bih, bhh)
    return scores


# speedup vs baseline: 3.4938x; 1.2897x over previous
import functools

import jax
import jax.numpy as jnp
from jax import lax
from jax.experimental import pallas as pl
from jax.experimental.pallas import tpu as pltpu
from jax.experimental.pallas import tpu_sc as plsc

NB = 50
K = 32
D = 128
DM = 256
RBLK = 128


CHUNK = 128


def _sc_gather(table, ent_idx, rel_idx, cen_idx):
    info = plsc.get_sparse_core_info()
    nc, ns = info.num_cores, info.num_subcores
    nw = nc * ns
    n_ent = ent_idx.shape[0]
    n_rel = rel_idx.shape[0]
    n_cen = cen_idx.shape[0]
    mesh = plsc.VectorSubcoreMesh(core_axis_name="c", subcore_axis_name="s")

    @functools.partial(
        pl.kernel, mesh=mesh,
        out_type=(
            jax.ShapeDtypeStruct((n_ent * CHUNK, D), jnp.float32),
            jax.ShapeDtypeStruct((n_rel * CHUNK, D), jnp.float32),
            jax.ShapeDtypeStruct((n_cen * CHUNK, D), jnp.float32),
        ),
        scratch_types=[
            pltpu.VMEM((CHUNK,), jnp.int32),
            pltpu.VMEM((CHUNK, D), jnp.float32),
            pltpu.SemaphoreType.DMA,
        ],
    )
    def k(table_h, ent_i, rel_i, cen_i, ent_o, rel_o, cen_o, idx_v, buf_v, sem):
        wid = lax.axis_index("s") * nc + lax.axis_index("c")

        def run(idx_h, out_h, n_chunks):
            n_mine = lax.max(0, (n_chunks - wid + nw - 1) // nw)

            def body(kk, carry):
                chunk = wid + kk * nw
                pltpu.sync_copy(idx_h.at[chunk], idx_v)
                pltpu.async_copy(table_h.at[idx_v], buf_v, sem).wait()
                pltpu.sync_copy(buf_v, out_h.at[pl.ds(chunk * CHUNK, CHUNK)])
                return carry

            lax.fori_loop(0, n_mine, body, 0)

        run(ent_i, ent_o, n_ent)
        run(rel_i, rel_o, n_rel)
        run(cen_i, cen_o, n_cen)

    return k(table, ent_idx, rel_idx, cen_idx)


def _encoder_block_kernel(ent_ref, rel_ref, cen_ref, w_ref, b_ref, out_ref):
    cen = cen_ref[...]
    R = cen.shape[0]

    d_cols, nn_cols = [], []
    for j in range(NB):
        e = ent_ref[:, j, :]
        d_cols.append(jnp.sum(cen * e, axis=-1, keepdims=True))
        nn_cols.append(jnp.sum(e * e, axis=-1, keepdims=True))
    d = jnp.concatenate(d_cols, axis=1)
    nn = jnp.concatenate(nn_cols, axis=1)
    cn = jnp.sum(cen * cen, axis=-1)
    an = jnp.maximum(jnp.sqrt(cn), 1e-8)[:, None]
    bn = jnp.maximum(jnp.sqrt(nn), 1e-8)
    sim = d / (an * bn)

    i_iota = jax.lax.broadcasted_iota(jnp.int32, (R, NB), 1)
    rank = jnp.zeros((R, NB), jnp.float32)
    for j in range(NB):
        sj = sim[:, j:j + 1]
        rank = rank + (sj > sim).astype(jnp.float32)
        rank = rank + ((sj == sim) & (j < i_iota)).astype(jnp.float32)
    w = jnp.where(rank < K, 1.0 / K, 0.0)

    m_rel = jnp.zeros((R, D), jnp.float32)
    m_ent = jnp.zeros((R, D), jnp.float32)
    for j in range(NB):
        wj = w[:, j:j + 1]
        m_rel = m_rel + wj * rel_ref[:, j, :]
        m_ent = m_ent + wj * ent_ref[:, j, :]
    mc = jnp.concatenate([m_rel, m_ent], axis=-1)
    pre = jnp.dot(mc, w_ref[...].T,
                  preferred_element_type=jnp.float32) + b_ref[...]
    out_ref[...] = jnp.tanh(pre)


def _head_kernel(neigh_ref, p1W_ref, p1b_ref, p2W_ref, p2b_ref,
                 ln_g_ref, ln_b_ref, Wih_ref, Whh_ref, bih_ref, bhh_ref,
                 out_ref, B):
    neigh = neigh_ref[...]
    qn = jnp.concatenate([neigh[0:B], neigh[B:2 * B]], axis=1)
    sn = jnp.concatenate([neigh[2 * B:2 * B + 5],
                          neigh[2 * B + 8:2 * B + 13]], axis=1)

    p1W = p1W_ref[...]
    p2W = p2W_ref[...]
    ln_g = ln_g_ref[...]
    ln_b = ln_b_ref[...]

    def enc(x):
        out = jax.nn.relu(jnp.dot(x, p1W.T, preferred_element_type=jnp.float32)
                          + p1b_ref[...])
        out = jnp.dot(out, p2W.T, preferred_element_type=jnp.float32) + p2b_ref[...]
        out = out + x
        m = jnp.mean(out, axis=-1, keepdims=True)
        v = jnp.mean((out - m) ** 2, axis=-1, keepdims=True)
        return (out - m) / jnp.sqrt(v + 1e-5) * ln_g + ln_b

    support_g = jnp.mean(enc(sn), axis=0, keepdims=True)
    query_g = enc(qn)

    Wih = Wih_ref[...]
    Whh = Whh_ref[...]
    bih = bih_ref[...]
    bhh = bhh_ref[...]
    sup_b = jnp.broadcast_to(support_g, (B, DM))

    h_r = jnp.zeros((B, 2 * DM), jnp.float32)
    c = jnp.zeros((B, 2 * DM), jnp.float32)
    h = query_g
    for _ in range(2):
        gates = (jnp.dot(query_g, Wih.T, preferred_element_type=jnp.float32)
                 + bih
                 + jnp.dot(h_r, Whh.T, preferred_element_type=jnp.float32)
                 + bhh)
        i_g = gates[:, 0:2 * DM]
        f_g = gates[:, 2 * DM:4 * DM]
        g_g = gates[:, 4 * DM:6 * DM]
        o_g = gates[:, 6 * DM:8 * DM]
        c = jax.nn.sigmoid(f_g) * c + jax.nn.sigmoid(i_g) * jnp.tanh(g_g)
        h_new = jax.nn.sigmoid(o_g) * jnp.tanh(c)
        h = query_g + h_new[:, :DM]
        h_r = jnp.concatenate([h, sup_b], axis=1)

    qf = h / jnp.maximum(jnp.linalg.norm(h, axis=-1, keepdims=True), 1e-12)
    sv = support_g[0]
    sv = sv / jnp.maximum(jnp.linalg.norm(sv), 1e-12)
    out_ref[...] = jnp.dot(qf, sv[:, None],
                           preferred_element_type=jnp.float32)[:, 0]


def kernel(query, support, q_l_conn, q_l_deg, q_r_conn, q_r_deg,
           s_l_conn, s_l_deg, s_r_conn, s_r_deg, table,
           gcn_wW, gcn_wb, gcn_b, p1W, p1b, p2W, p2b, ln_g, ln_b,
           Wih, Whh, bih, bhh):
    B = query.shape[0]
    FEW = support.shape[0]

    n_rows = 2 * B + 16
    rows_pad = ((n_rows + RBLK - 1) // RBLK) * RBLK
    ids = jnp.zeros((rows_pad,), jnp.int32)
    ids = ids.at[0:B].set(query[:, 0].astype(jnp.int32))
    ids = ids.at[B:2 * B].set(query[:, 1].astype(jnp.int32))
    ids = ids.at[2 * B:2 * B + FEW].set(support[:, 0].astype(jnp.int32))
    ids = ids.at[2 * B + 8:2 * B + 8 + FEW].set(support[:, 1].astype(jnp.int32))
    conn = jnp.zeros((rows_pad, NB, 2), jnp.int32)
    conn = conn.at[0:B].set(q_l_conn.astype(jnp.int32))
    conn = conn.at[B:2 * B].set(q_r_conn.astype(jnp.int32))
    conn = conn.at[2 * B:2 * B + FEW].set(s_l_conn.astype(jnp.int32))
    conn = conn.at[2 * B + 8:2 * B + 8 + FEW].set(s_r_conn.astype(jnp.int32))

    ent_idx = conn[:, :, 1].reshape(rows_pad * NB // CHUNK, CHUNK)
    rel_idx = conn[:, :, 0].reshape(rows_pad * NB // CHUNK, CHUNK)
    cen_idx = ids.reshape(rows_pad // CHUNK, CHUNK)
    ent_f, rel_f, cen_g = _sc_gather(table, ent_idx, rel_idx, cen_idx)
    ent_g = ent_f.reshape(rows_pad, NB, D)
    rel_g = rel_f.reshape(rows_pad, NB, D)

    nblk = rows_pad // RBLK
    neigh = pl.pallas_call(
        _encoder_block_kernel,
        grid=(nblk,),
        in_specs=[
            pl.BlockSpec((RBLK, NB, D), lambda i: (i, 0, 0)),
            pl.BlockSpec((RBLK, NB, D), lambda i: (i, 0, 0)),
            pl.BlockSpec((RBLK, D), lambda i: (i, 0)),
            pl.BlockSpec((D, 2 * D), lambda i: (0, 0)),
            pl.BlockSpec((D,), lambda i: (0,)),
        ],
        out_specs=pl.BlockSpec((RBLK, D), lambda i: (i, 0)),
        out_shape=jax.ShapeDtypeStruct((rows_pad, D), jnp.float32),
    )(ent_g, rel_g, cen_g, gcn_wW, gcn_wb + gcn_b)

    scores = pl.pallas_call(
        functools.partial(_head_kernel, B=B),
        out_shape=jax.ShapeDtypeStruct((B,), jnp.float32),
        compiler_params=pltpu.CompilerParams(
            vmem_limit_bytes=63 * 1024 * 1024),
    )(neigh, p1W, p1b, p2W, p2b, ln_g, ln_b, Wih, Whh, bih, bhh)
    return scores


# speedup vs baseline: 3.5466x; 1.0151x over previous
import functools

import jax
import jax.numpy as jnp
from jax import lax
from jax.experimental import pallas as pl
from jax.experimental.pallas import tpu as pltpu
from jax.experimental.pallas import tpu_sc as plsc

NB = 50
K = 32
D = 128
DM = 256
RBLK = 128


CHUNK = 128


NSLOT = 4


def _sc_gather(table, ent_idx, rel_idx, cen_idx):
    info = plsc.get_sparse_core_info()
    nc, ns = info.num_cores, info.num_subcores
    nw = nc * ns
    cpw_ent = ent_idx.shape[1]
    cpw_rel = rel_idx.shape[1]
    cpw_cen = cen_idx.shape[1]
    n_ent, n_rel, n_cen = 850, 850, 17
    mesh = plsc.VectorSubcoreMesh(core_axis_name="c", subcore_axis_name="s")

    @functools.partial(
        pl.kernel, mesh=mesh,
        out_type=(
            jax.ShapeDtypeStruct((n_ent * CHUNK, D), jnp.float32),
            jax.ShapeDtypeStruct((n_rel * CHUNK, D), jnp.float32),
            jax.ShapeDtypeStruct((n_cen * CHUNK, D), jnp.float32),
        ),
        scratch_types=(
            [pltpu.VMEM((max(cpw_ent, cpw_rel, cpw_cen), CHUNK), jnp.int32)]
            + [pltpu.VMEM((CHUNK, D), jnp.float32) for _ in range(NSLOT)]
            + [pltpu.SemaphoreType.DMA for _ in range(2 * NSLOT + 1)]
        ),
    )
    def k(table_h, ent_i, rel_i, cen_i, ent_o, rel_o, cen_o,
          idx_v, *bufs_and_sems):
        bufs = bufs_and_sems[:NSLOT]
        gsems = bufs_and_sems[NSLOT:2 * NSLOT]
        wsems = bufs_and_sems[2 * NSLOT:3 * NSLOT]
        isem = bufs_and_sems[3 * NSLOT]
        wid = lax.axis_index("s") * nc + lax.axis_index("c")

        def run(idx_h, out_h, cpw, total):
            base = wid * cpw
            pltpu.async_copy(idx_h.at[wid], idx_v.at[pl.ds(0, cpw)],
                             isem).wait()
            def wait_write(b):
                pltpu.make_async_copy(
                    bufs[b], out_h.at[pl.ds(0, CHUNK)], wsems[b]).wait()

            nrounds = (cpw + NSLOT - 1) // NSLOT
            for t in range(nrounds):
                for b in range(NSLOT):
                    kk = t * NSLOT + b
                    if kk >= cpw:
                        continue
                    chunk = base + kk
                    if t > 0:
                        @pl.when(base + (t - 1) * NSLOT + b < total)
                        def _(b=b):
                            wait_write(b)
                    @pl.when(chunk < total)
                    def _(kk=kk, b=b):
                        pltpu.async_copy(
                            table_h.at[idx_v.at[kk]], bufs[b], gsems[b])
                for b in range(NSLOT):
                    kk = t * NSLOT + b
                    if kk >= cpw:
                        continue
                    chunk = base + kk
                    @pl.when(chunk < total)
                    def _(kk=kk, b=b, chunk=chunk):
                        pltpu.make_async_copy(
                            table_h.at[idx_v.at[kk]], bufs[b],
                            gsems[b]).wait()
                        pltpu.async_copy(
                            bufs[b], out_h.at[pl.ds(chunk * CHUNK, CHUNK)],
                            wsems[b])
            for b in range(NSLOT):
                if b >= cpw:
                    continue
                t_last = (cpw - 1 - b) // NSLOT
                @pl.when(base + t_last * NSLOT + b < total)
                def _(b=b):
                    wait_write(b)

        run(ent_i, ent_o, cpw_ent, n_ent)
        run(rel_i, rel_o, cpw_rel, n_rel)
        run(cen_i, cen_o, cpw_cen, n_cen)

    return k(table, ent_idx, rel_idx, cen_idx)


def _encoder_block_kernel(ent_ref, rel_ref, cen_ref, w_ref, b_ref, out_ref):
    cen = cen_ref[...]
    R = cen.shape[0]

    d_cols, nn_cols = [], []
    for j in range(NB):
        e = ent_ref[:, j, :]
        d_cols.append(jnp.sum(cen * e, axis=-1, keepdims=True))
        nn_cols.append(jnp.sum(e * e, axis=-1, keepdims=True))
    d = jnp.concatenate(d_cols, axis=1)
    nn = jnp.concatenate(nn_cols, axis=1)
    cn = jnp.sum(cen * cen, axis=-1)
    an = jnp.maximum(jnp.sqrt(cn), 1e-8)[:, None]
    bn = jnp.maximum(jnp.sqrt(nn), 1e-8)
    sim = d / (an * bn)

    i_iota = jax.lax.broadcasted_iota(jnp.int32, (R, NB), 1)
    rank = jnp.zeros((R, NB), jnp.float32)
    for j in range(NB):
        sj = sim[:, j:j + 1]
        rank = rank + (sj > sim).astype(jnp.float32)
        rank = rank + ((sj == sim) & (j < i_iota)).astype(jnp.float32)
    w = jnp.where(rank < K, 1.0 / K, 0.0)

    m_rel = jnp.zeros((R, D), jnp.float32)
    m_ent = jnp.zeros((R, D), jnp.float32)
    for j in range(NB):
        wj = w[:, j:j + 1]
        m_rel = m_rel + wj * rel_ref[:, j, :]
        m_ent = m_ent + wj * ent_ref[:, j, :]
    mc = jnp.concatenate([m_rel, m_ent], axis=-1)
    pre = jnp.dot(mc, w_ref[...].T,
                  preferred_element_type=jnp.float32) + b_ref[...]
    out_ref[...] = jnp.tanh(pre)


def _head_kernel(neigh_ref, p1W_ref, p1b_ref, p2W_ref, p2b_ref,
                 ln_g_ref, ln_b_ref, Wih_ref, Whh_ref, bih_ref, bhh_ref,
                 out_ref, B):
    neigh = neigh_ref[...]
    qn = jnp.concatenate([neigh[0:B], neigh[B:2 * B]], axis=1)
    sn = jnp.concatenate([neigh[2 * B:2 * B + 5],
                          neigh[2 * B + 8:2 * B + 13]], axis=1)

    p1W = p1W_ref[...]
    p2W = p2W_ref[...]
    ln_g = ln_g_ref[...]
    ln_b = ln_b_ref[...]

    def enc(x):
        out = jax.nn.relu(jnp.dot(x, p1W.T, preferred_element_type=jnp.float32)
                          + p1b_ref[...])
        out = jnp.dot(out, p2W.T, preferred_element_type=jnp.float32) + p2b_ref[...]
        out = out + x
        m = jnp.mean(out, axis=-1, keepdims=True)
        v = jnp.mean((out - m) ** 2, axis=-1, keepdims=True)
        return (out - m) / jnp.sqrt(v + 1e-5) * ln_g + ln_b

    support_g = jnp.mean(enc(sn), axis=0, keepdims=True)
    query_g = enc(qn)

    Wih = Wih_ref[...]
    Whh = Whh_ref[...]
    bih = bih_ref[...]
    bhh = bhh_ref[...]
    sup_b = jnp.broadcast_to(support_g, (B, DM))

    h_r = jnp.zeros((B, 2 * DM), jnp.float32)
    c = jnp.zeros((B, 2 * DM), jnp.float32)
    h = query_g
    for _ in range(2):
        gates = (jnp.dot(query_g, Wih.T, preferred_element_type=jnp.float32)
                 + bih
                 + jnp.dot(h_r, Whh.T, preferred_element_type=jnp.float32)
                 + bhh)
        i_g = gates[:, 0:2 * DM]
        f_g = gates[:, 2 * DM:4 * DM]
        g_g = gates[:, 4 * DM:6 * DM]
        o_g = gates[:, 6 * DM:8 * DM]
        c = jax.nn.sigmoid(f_g) * c + jax.nn.sigmoid(i_g) * jnp.tanh(g_g)
        h_new = jax.nn.sigmoid(o_g) * jnp.tanh(c)
        h = query_g + h_new[:, :DM]
        h_r = jnp.concatenate([h, sup_b], axis=1)

    qf = h / jnp.maximum(jnp.linalg.norm(h, axis=-1, keepdims=True), 1e-12)
    sv = support_g[0]
    sv = sv / jnp.maximum(jnp.linalg.norm(sv), 1e-12)
    out_ref[...] = jnp.dot(qf, sv[:, None],
                           preferred_element_type=jnp.float32)[:, 0]


def kernel(query, support, q_l_conn, q_l_deg, q_r_conn, q_r_deg,
           s_l_conn, s_l_deg, s_r_conn, s_r_deg, table,
           gcn_wW, gcn_wb, gcn_b, p1W, p1b, p2W, p2b, ln_g, ln_b,
           Wih, Whh, bih, bhh):
    B = query.shape[0]
    FEW = support.shape[0]

    n_rows = 2 * B + 16
    rows_pad = ((n_rows + RBLK - 1) // RBLK) * RBLK
    ids = jnp.zeros((rows_pad,), jnp.int32)
    ids = ids.at[0:B].set(query[:, 0].astype(jnp.int32))
    ids = ids.at[B:2 * B].set(query[:, 1].astype(jnp.int32))
    ids = ids.at[2 * B:2 * B + FEW].set(support[:, 0].astype(jnp.int32))
    ids = ids.at[2 * B + 8:2 * B + 8 + FEW].set(support[:, 1].astype(jnp.int32))
    conn = jnp.zeros((rows_pad, NB, 2), jnp.int32)
    conn = conn.at[0:B].set(q_l_conn.astype(jnp.int32))
    conn = conn.at[B:2 * B].set(q_r_conn.astype(jnp.int32))
    conn = conn.at[2 * B:2 * B + FEW].set(s_l_conn.astype(jnp.int32))
    conn = conn.at[2 * B + 8:2 * B + 8 + FEW].set(s_r_conn.astype(jnp.int32))

    def _chunked(flat, cpw):
        pad = 32 * cpw * CHUNK - flat.shape[0]
        return jnp.pad(flat, (0, pad)).reshape(32, cpw, CHUNK)

    ent_idx = _chunked(conn[:, :, 1].reshape(-1), 27)
    rel_idx = _chunked(conn[:, :, 0].reshape(-1), 27)
    cen_idx = _chunked(ids, 1)
    ent_f, rel_f, cen_g = _sc_gather(table, ent_idx, rel_idx, cen_idx)
    ent_g = ent_f.reshape(rows_pad, NB, D)
    rel_g = rel_f.reshape(rows_pad, NB, D)

    nblk = rows_pad // RBLK
    neigh = pl.pallas_call(
        _encoder_block_kernel,
        grid=(nblk,),
        in_specs=[
            pl.BlockSpec((RBLK, NB, D), lambda i: (i, 0, 0)),
            pl.BlockSpec((RBLK, NB, D), lambda i: (i, 0, 0)),
            pl.BlockSpec((RBLK, D), lambda i: (i, 0)),
            pl.BlockSpec((D, 2 * D), lambda i: (0, 0)),
            pl.BlockSpec((D,), lambda i: (0,)),
        ],
        out_specs=pl.BlockSpec((RBLK, D), lambda i: (i, 0)),
        out_shape=jax.ShapeDtypeStruct((rows_pad, D), jnp.float32),
    )(ent_g, rel_g, cen_g, gcn_wW, gcn_wb + gcn_b)

    scores = pl.pallas_call(
        functools.partial(_head_kernel, B=B),
        out_shape=jax.ShapeDtypeStruct((B,), jnp.float32),
        compiler_params=pltpu.CompilerParams(
            vmem_limit_bytes=63 * 1024 * 1024),
    )(neigh, p1W, p1b, p2W, p2b, ln_g, ln_b, Wih, Whh, bih, bhh)
    return scores


# speedup vs baseline: 3.6073x; 1.0171x over previous
import functools

import jax
import jax.numpy as jnp
from jax import lax
from jax.experimental import pallas as pl
from jax.experimental.pallas import tpu as pltpu
from jax.experimental.pallas import tpu_sc as plsc

NB = 50
K = 32
D = 128
DM = 256
ROWS = 2176
RPW = ROWS // 32
W1 = 56
NEG = -3.0e38


def _rsqrt_newton(x):
    xi = plsc.bitcast(x, jnp.int32)
    yi = 0x5F3759DF - lax.shift_right_logical(xi, 1)
    y = plsc.bitcast(yi, jnp.float32)
    for _ in range(3):
        y = y * (1.5 - 0.5 * x * y * y)
    return y


def _sort16(x):
    return jnp.sort(x)


def _merge2(a, b):
    rb = jnp.flip(b, 0)
    lo = jnp.minimum(a, rb)
    hi = jnp.maximum(a, rb)
    return _sort16(lo), _sort16(hi)


def _bmerge32(p, q):
    lo = jnp.minimum(p, q)
    hi = jnp.maximum(p, q)
    return _sort16(lo), _sort16(hi)


def _sc_encode(table, idx1, relids):
    mesh = plsc.VectorSubcoreMesh(core_axis_name="c", subcore_axis_name="s")

    @functools.partial(
        pl.kernel, mesh=mesh,
        compiler_params=pltpu.CompilerParams(needs_layout_passes=False),
        out_type=jax.ShapeDtypeStruct((32, RPW, 2 * D), jnp.float32),
        scratch_types=(
            [pltpu.VMEM((RPW, W1), jnp.int32),
             pltpu.VMEM((RPW, 64), jnp.int32),
             pltpu.VMEM((W1, D), jnp.float32),
             pltpu.VMEM((W1, D), jnp.float32),
             pltpu.VMEM((K, D), jnp.float32),
             pltpu.VMEM((K, D), jnp.float32),
             pltpu.VMEM((K,), jnp.int32),
             pltpu.VMEM((K,), jnp.int32),
             pltpu.VMEM((64,), jnp.float32),
             pltpu.VMEM((64,), jnp.float32),
             pltpu.VMEM((RPW, 2 * D), jnp.float32)]
            + [pltpu.SemaphoreType.DMA for _ in range(6)]
        ),
    )
    def k(table_h, idx1_h, relids_h, mean_o,
          idx1_v, relids_v, bufA0, bufA1, bufB0, bufB1, rs0, rs1,
          d_buf, nn_buf, out_v, psem, ga0, ga1, gb0, gb1, wsem):
        nc = 2
        wid = lax.axis_index("s") * nc + lax.axis_index("c")
        base = wid * RPW
        bufA = (bufA0, bufA1)
        bufB = (bufB0, bufB1)
        rs = (rs0, rs1)
        ga = (ga0, ga1)
        gb = (gb0, gb1)

        pltpu.async_copy(idx1_h.at[wid], idx1_v, psem).wait()
        pltpu.async_copy(relids_h.at[wid], relids_v, psem).wait()

        iota = lax.iota(jnp.int32, 16)
        jrow = [jnp.where((g * 16 + iota) < NB, 1 + g * 16 + iota, 0)
                for g in range(4)]
        valid3 = iota < (NB - 48)
        c32 = jnp.full((16,), K, jnp.int32)

        def issue_a(r, s):
            pltpu.async_copy(table_h.at[idx1_v.at[r]], bufA[s], ga[s])

        issue_a(0, 0)

        def substep(t, b):
            r = 2 * t + b
            s = b
            o = 1 - b

            pltpu.make_async_copy(
                table_h.at[idx1_v.at[r]], bufA[s], ga[s]).wait()

            @pl.when(r + 1 < RPW)
            def _():
                issue_a(r + 1, o)

            zero = jnp.zeros((16,), jnp.float32)
            cenc = [bufA[s][0, 16 * c:16 * c + 16] for c in range(8)]
            lane0 = iota == 0

            def jbody(j, carry):
                da = zero
                na = zero
                for c in range(8):
                    e = bufA[s][1 + j, 16 * c:16 * c + 16]
                    da = da + cenc[c] * e
                    na = na + e * e
                jv = jnp.full((16,), 0, jnp.int32) + j
                plsc.store_scatter(
                    d_buf, [jv], jnp.broadcast_to(jnp.sum(da), (16,)),
                    mask=lane0)
                plsc.store_scatter(
                    nn_buf, [jv], jnp.broadcast_to(jnp.sum(na), (16,)),
                    mask=lane0)
                return carry

            lax.fori_loop(0, NB, jbody, 0)
            keys = []
            for g in range(4):
                d_g = d_buf[16 * g:16 * g + 16]
                n_g = nn_buf[16 * g:16 * g + 16]
                kg = d_g * _rsqrt_newton(jnp.maximum(n_g, 1e-16))
                if g == 3:
                    kg = jnp.where(valid3, kg, NEG)
                keys.append(kg)

            s0, s1, s2, s3 = (_sort16(x) for x in keys)
            a0, a1 = _merge2(s0, s1)
            b0, b1 = _merge2(s2, s3)
            ry0, ry1 = jnp.flip(b1, 0), jnp.flip(b0, 0)
            l0, l1 = jnp.minimum(a0, ry0), jnp.minimum(a1, ry1)
            h0, h1 = jnp.maximum(a0, ry0), jnp.maximum(a1, ry1)
            _, _ = _bmerge32(l0, l1)
            z2, _ = _bmerge32(h0, h1)
            t_thr = jnp.broadcast_to(jnp.min(z2), (16,))

            gts = [kg > t_thr for kg in keys]
            c_gt = jnp.zeros((16,), jnp.int32)
            for g in range(4):
                c_gt = c_gt + plsc.all_reduce_population_count(gts[g])
            allow = c32 - c_gt
            prior = jnp.zeros((16,), jnp.int32)
            selprior = jnp.zeros((16,), jnp.int32)
            sels = []
            for g in range(4):
                eq = keys[g] == t_thr
                inc = plsc.cumsum(eq.astype(jnp.int32))
                take = eq & ((inc + prior) <= allow)
                prior = prior + plsc.all_reduce_population_count(eq)
                sel = gts[g] | take
                sels.append(sel)
                sel_i = sel.astype(jnp.int32)
                pos = plsc.cumsum(sel_i) - sel_i + selprior
                selprior = selprior + plsc.all_reduce_population_count(sel)
                rid = relids_v[r, g * 16:(g + 1) * 16]
                plsc.store_scatter(rs[s], [pos], rid, mask=sel)

            pltpu.async_copy(table_h.at[rs[s]], bufB[s], gb[s])

            wg = [sels[g].astype(jnp.float32) for g in range(4)]
            eacc = [zero] * 8
            for j in range(NB):
                g, i = j // 16, j % 16
                oh = (iota == i).astype(jnp.float32)
                wj = jnp.sum(wg[g] * oh)
                for c in range(8):
                    eacc[c] = eacc[c] + wj * bufA[s][1 + j,
                                                     16 * c:16 * c + 16]
            for c in range(8):
                out_v[r, D + 16 * c:D + 16 * c + 16] = eacc[c] * (1.0 / K)

            @pl.when(r >= 1)
            def _():
                pltpu.make_async_copy(
                    table_h.at[rs[o]], bufB[o], gb[o]).wait()

                racc = [zero] * 8
                for j in range(K):
                    for c in range(8):
                        racc[c] = racc[c] + bufB[o][j, 16 * c:16 * c + 16]
                for c in range(8):
                    out_v[r - 1, 16 * c:16 * c + 16] = racc[c] * (1.0 / K)

        def pair(t, carry):
            substep(t, 0)
            substep(t, 1)
            return carry

        lax.fori_loop(0, RPW // 2, pair, 0)

        pltpu.make_async_copy(table_h.at[rs[1]], bufB[1], gb[1]).wait()

        racc = [jnp.zeros((16,), jnp.float32)] * 8
        for j in range(K):
            for c in range(8):
                racc[c] = racc[c] + bufB[1][j, 16 * c:16 * c + 16]
        for c in range(8):
            out_v[RPW - 1, 16 * c:16 * c + 16] = racc[c] * (1.0 / K)
        pltpu.async_copy(out_v, mean_o.at[wid], wsem).wait()

    return k(table, idx1.reshape(32, RPW, W1),
             relids.reshape(32, RPW, 64)).reshape(ROWS, 2 * D)


def _head_kernel(mean_ref, gcnW_ref, gcnb_ref, p1W_ref, p1b_ref, p2W_ref,
                 p2b_ref, ln_g_ref, ln_b_ref, Wih_ref, Whh_ref, bih_ref,
                 bhh_ref, out_ref, B):
    mc = mean_ref[...]
    neigh = jnp.tanh(jnp.dot(mc, gcnW_ref[...].T,
                             preferred_element_type=jnp.float32)
                     + gcnb_ref[...])
    qn = jnp.concatenate([neigh[0:B], neigh[B:2 * B]], axis=1)
    sn = jnp.concatenate([neigh[2 * B:2 * B + 5],
                          neigh[2 * B + 8:2 * B + 13]], axis=1)

    p1W = p1W_ref[...]
    p2W = p2W_ref[...]
    ln_g = ln_g_ref[...]
    ln_b = ln_b_ref[...]

    def enc(x):
        out = jax.nn.relu(jnp.dot(x, p1W.T, preferred_element_type=jnp.float32)
                          + p1b_ref[...])
        out = jnp.dot(out, p2W.T, preferred_element_type=jnp.float32) + p2b_ref[...]
        out = out + x
        m = jnp.mean(out, axis=-1, keepdims=True)
        v = jnp.mean((out - m) ** 2, axis=-1, keepdims=True)
        return (out - m) / jnp.sqrt(v + 1e-5) * ln_g + ln_b

    support_g = jnp.mean(enc(sn), axis=0, keepdims=True)
    query_g = enc(qn)

    Wih = Wih_ref[...]
    Whh = Whh_ref[...]
    bih = bih_ref[...]
    bhh = bhh_ref[...]
    sup_b = jnp.broadcast_to(support_g, (B, DM))

    h_r = jnp.zeros((B, 2 * DM), jnp.float32)
    c = jnp.zeros((B, 2 * DM), jnp.float32)
    h = query_g
    for _ in range(2):
        gates = (jnp.dot(query_g, Wih.T, preferred_element_type=jnp.float32)
                 + bih
                 + jnp.dot(h_r, Whh.T, preferred_element_type=jnp.float32)
                 + bhh)
        i_g = gates[:, 0:2 * DM]
        f_g = gates[:, 2 * DM:4 * DM]
        g_g = gates[:, 4 * DM:6 * DM]
        o_g = gates[:, 6 * DM:8 * DM]
        c = jax.nn.sigmoid(f_g) * c + jax.nn.sigmoid(i_g) * jnp.tanh(g_g)
        h_new = jax.nn.sigmoid(o_g) * jnp.tanh(c)
        h = query_g + h_new[:, :DM]
        h_r = jnp.concatenate([h, sup_b], axis=1)

    qf = h / jnp.maximum(jnp.linalg.norm(h, axis=-1, keepdims=True), 1e-12)
    sv = support_g[0]
    sv = sv / jnp.maximum(jnp.linalg.norm(sv), 1e-12)
    out_ref[...] = jnp.dot(qf, sv[:, None],
                           preferred_element_type=jnp.float32)[:, 0]


def kernel(query, support, q_l_conn, q_l_deg, q_r_conn, q_r_deg,
           s_l_conn, s_l_deg, s_r_conn, s_r_deg, table,
           gcn_wW, gcn_wb, gcn_b, p1W, p1b, p2W, p2b, ln_g, ln_b,
           Wih, Whh, bih, bhh):
    B = query.shape[0]
    FEW = support.shape[0]

    ids = jnp.zeros((ROWS,), jnp.int32)
    ids = ids.at[0:B].set(query[:, 0].astype(jnp.int32))
    ids = ids.at[B:2 * B].set(query[:, 1].astype(jnp.int32))
    ids = ids.at[2 * B:2 * B + FEW].set(support[:, 0].astype(jnp.int32))
    ids = ids.at[2 * B + 8:2 * B + 8 + FEW].set(support[:, 1].astype(jnp.int32))
    conn = jnp.zeros((ROWS, NB, 2), jnp.int32)
    conn = conn.at[0:B].set(q_l_conn.astype(jnp.int32))
    conn = conn.at[B:2 * B].set(q_r_conn.astype(jnp.int32))
    conn = conn.at[2 * B:2 * B + FEW].set(s_l_conn.astype(jnp.int32))
    conn = conn.at[2 * B + 8:2 * B + 8 + FEW].set(s_r_conn.astype(jnp.int32))

    idx1 = jnp.concatenate(
        [ids[:, None], conn[:, :, 1],
         jnp.zeros((ROWS, W1 - 1 - NB), jnp.int32)], axis=1)
    relids = jnp.concatenate(
        [conn[:, :, 0], jnp.zeros((ROWS, 64 - NB), jnp.int32)], axis=1)

    mean = _sc_encode(table, idx1, relids)

    scores = pl.pallas_call(
        functools.partial(_head_kernel, B=B),
        out_shape=jax.ShapeDtypeStruct((B,), jnp.float32),
        compiler_params=pltpu.CompilerParams(
            vmem_limit_bytes=63 * 1024 * 1024),
    )(mean, gcn_wW, gcn_wb + gcn_b, p1W, p1b, p2W, p2b, ln_g, ln_b,
      Wih, Whh, bih, bhh)
    return scores


# speedup vs baseline: 3.6185x; 1.0031x over previous
import functools

import jax
import jax.numpy as jnp
from jax import lax
from jax.experimental import pallas as pl
from jax.experimental.pallas import tpu as pltpu
from jax.experimental.pallas import tpu_sc as plsc

NB = 50
K = 32
D = 128
DM = 256
ROWS = 2176
RPW = ROWS // 32
W1 = 56
NEG = -3.0e38


def _rsqrt_newton(x):
    xi = plsc.bitcast(x, jnp.int32)
    yi = 0x5F3759DF - lax.shift_right_logical(xi, 1)
    y = plsc.bitcast(yi, jnp.float32)
    for _ in range(3):
        y = y * (1.5 - 0.5 * x * y * y)
    return y


def _sort16(x):
    return jnp.sort(x)


def _merge2(a, b):
    rb = jnp.flip(b, 0)
    lo = jnp.minimum(a, rb)
    hi = jnp.maximum(a, rb)
    return _sort16(lo), _sort16(hi)


def _bmerge32(p, q):
    lo = jnp.minimum(p, q)
    hi = jnp.maximum(p, q)
    return _sort16(lo), _sort16(hi)


def _sc_encode(table, idx1, relids):
    mesh = plsc.VectorSubcoreMesh(core_axis_name="c", subcore_axis_name="s")

    @functools.partial(
        pl.kernel, mesh=mesh,
        compiler_params=pltpu.CompilerParams(needs_layout_passes=False),
        out_type=jax.ShapeDtypeStruct((32, RPW, 2 * D), jnp.float32),
        scratch_types=(
            [pltpu.VMEM((RPW, W1), jnp.int32),
             pltpu.VMEM((RPW, 64), jnp.int32),
             pltpu.VMEM((W1, D), jnp.float32),
             pltpu.VMEM((W1, D), jnp.float32),
             pltpu.VMEM((K, D), jnp.float32),
             pltpu.VMEM((K, D), jnp.float32),
             pltpu.VMEM((K,), jnp.int32),
             pltpu.VMEM((K,), jnp.int32),
             pltpu.VMEM((64,), jnp.float32),
             pltpu.VMEM((RPW, 2 * D), jnp.float32)]
            + [pltpu.SemaphoreType.DMA for _ in range(6)]
        ),
    )
    def k(table_h, idx1_h, relids_h, mean_o,
          idx1_v, relids_v, bufA0, bufA1, bufB0, bufB1, rs0, rs1,
          w_buf, out_v, psem, ga0, ga1, gb0, gb1, wsem):
        nc = 2
        wid = lax.axis_index("s") * nc + lax.axis_index("c")
        base = wid * RPW
        bufA = (bufA0, bufA1)
        bufB = (bufB0, bufB1)
        rs = (rs0, rs1)
        ga = (ga0, ga1)
        gb = (gb0, gb1)

        pltpu.async_copy(idx1_h.at[wid], idx1_v, psem).wait()
        pltpu.async_copy(relids_h.at[wid], relids_v, psem).wait()

        iota = lax.iota(jnp.int32, 16)
        jrow = [jnp.where((g * 16 + iota) < NB, 1 + g * 16 + iota, 0)
                for g in range(4)]
        valid3 = iota < (NB - 48)
        c32 = jnp.full((16,), K, jnp.int32)

        def issue_a(r, s):
            pltpu.async_copy(table_h.at[idx1_v.at[r]], bufA[s], ga[s])

        issue_a(0, 0)

        def substep(t, b):
            r = 2 * t + b
            s = b
            o = 1 - b

            pltpu.make_async_copy(
                table_h.at[idx1_v.at[r]], bufA[s], ga[s]).wait()

            @pl.when(r + 1 < RPW)
            def _():
                issue_a(r + 1, o)

            zero = jnp.zeros((16,), jnp.float32)
            zrow = jnp.zeros((16,), jnp.int32)

            def fbody(f8, carry):
                accs = list(carry)
                for i in range(8):
                    fv = jnp.full((16,), i, jnp.int32) + f8 * 8
                    cf = plsc.load_gather(bufA[s], [zrow, fv])
                    for g in range(4):
                        col = plsc.load_gather(bufA[s], [jrow[g], fv])
                        accs[2 * g] = accs[2 * g] + cf * col
                        accs[2 * g + 1] = accs[2 * g + 1] + col * col
                return tuple(accs)

            accs = lax.fori_loop(0, 16, fbody, (zero,) * 8)
            keys = []
            for g in range(4):
                d_g, n_g = accs[2 * g], accs[2 * g + 1]
                kg = d_g * _rsqrt_newton(jnp.maximum(n_g, 1e-16))
                if g == 3:
                    kg = jnp.where(valid3, kg, NEG)
                keys.append(kg)

            s0, s1, s2, s3 = (_sort16(x) for x in keys)
            a0, a1 = _merge2(s0, s1)
            b0, b1 = _merge2(s2, s3)
            ry0, ry1 = jnp.flip(b1, 0), jnp.flip(b0, 0)
            l0, l1 = jnp.minimum(a0, ry0), jnp.minimum(a1, ry1)
            h0, h1 = jnp.maximum(a0, ry0), jnp.maximum(a1, ry1)
            _, _ = _bmerge32(l0, l1)
            z2, _ = _bmerge32(h0, h1)
            t_thr = jnp.broadcast_to(jnp.min(z2), (16,))

            gts = [kg > t_thr for kg in keys]
            c_gt = jnp.zeros((16,), jnp.int32)
            for g in range(4):
                c_gt = c_gt + plsc.all_reduce_population_count(gts[g])
            allow = c32 - c_gt
            prior = jnp.zeros((16,), jnp.int32)
            selprior = jnp.zeros((16,), jnp.int32)
            sels = []
            for g in range(4):
                eq = keys[g] == t_thr
                inc = plsc.cumsum(eq.astype(jnp.int32))
                take = eq & ((inc + prior) <= allow)
                prior = prior + plsc.all_reduce_population_count(eq)
                sel = gts[g] | take
                sels.append(sel)
                sel_i = sel.astype(jnp.int32)
                pos = plsc.cumsum(sel_i) - sel_i + selprior
                selprior = selprior + plsc.all_reduce_population_count(sel)
                rid = relids_v[r, g * 16:(g + 1) * 16]
                plsc.store_scatter(rs[s], [pos], rid, mask=sel)

            pltpu.async_copy(table_h.at[rs[s]], bufB[s], gb[s])

            for g in range(4):
                w_buf[16 * g:16 * g + 16] = sels[g].astype(jnp.float32)

            def wbody(j5, carry):
                accs = list(carry)
                for i in range(5):
                    wj = plsc.load_gather(
                        w_buf, [jnp.full((16,), i, jnp.int32) + j5 * 5])
                    for c in range(8):
                        accs[c] = accs[c] + wj * bufA[s][1 + j5 * 5 + i,
                                                         16 * c:16 * c + 16]
                return tuple(accs)

            eacc = lax.fori_loop(0, 10, wbody, (zero,) * 8)
            for c in range(8):
                out_v[r, D + 16 * c:D + 16 * c + 16] = eacc[c] * (1.0 / K)

            @pl.when(r >= 1)
            def _():
                pltpu.make_async_copy(
                    table_h.at[rs[o]], bufB[o], gb[o]).wait()

                racc = [zero] * 8
                for j in range(K):
                    for c in range(8):
                        racc[c] = racc[c] + bufB[o][j, 16 * c:16 * c + 16]
                for c in range(8):
                    out_v[r - 1, 16 * c:16 * c + 16] = racc[c] * (1.0 / K)

        def pair(t, carry):
            substep(t, 0)
            substep(t, 1)
            return carry

        lax.fori_loop(0, RPW // 2, pair, 0)

        pltpu.make_async_copy(table_h.at[rs[1]], bufB[1], gb[1]).wait()

        racc = [jnp.zeros((16,), jnp.float32)] * 8
        for j in range(K):
            for c in range(8):
                racc[c] = racc[c] + bufB[1][j, 16 * c:16 * c + 16]
        for c in range(8):
            out_v[RPW - 1, 16 * c:16 * c + 16] = racc[c] * (1.0 / K)
        pltpu.async_copy(out_v, mean_o.at[wid], wsem).wait()

    return k(table, idx1.reshape(32, RPW, W1),
             relids.reshape(32, RPW, 64)).reshape(ROWS, 2 * D)


def _head_kernel(mean_ref, gcnW_ref, gcnb_ref, p1W_ref, p1b_ref, p2W_ref,
                 p2b_ref, ln_g_ref, ln_b_ref, Wih_ref, Whh_ref, bih_ref,
                 bhh_ref, out_ref, B):
    mc = mean_ref[...]
    neigh = jnp.tanh(jnp.dot(mc, gcnW_ref[...].T,
                             preferred_element_type=jnp.float32)
                     + gcnb_ref[...])
    qn = jnp.concatenate([neigh[0:B], neigh[B:2 * B]], axis=1)
    sn = jnp.concatenate([neigh[2 * B:2 * B + 5],
                          neigh[2 * B + 8:2 * B + 13]], axis=1)

    p1W = p1W_ref[...]
    p2W = p2W_ref[...]
    ln_g = ln_g_ref[...]
    ln_b = ln_b_ref[...]

    def enc(x):
        out = jax.nn.relu(jnp.dot(x, p1W.T, preferred_element_type=jnp.float32)
                          + p1b_ref[...])
        out = jnp.dot(out, p2W.T, preferred_element_type=jnp.float32) + p2b_ref[...]
        out = out + x
        m = jnp.mean(out, axis=-1, keepdims=True)
        v = jnp.mean((out - m) ** 2, axis=-1, keepdims=True)
        return (out - m) / jnp.sqrt(v + 1e-5) * ln_g + ln_b

    support_g = jnp.mean(enc(sn), axis=0, keepdims=True)
    query_g = enc(qn)

    Wih = Wih_ref[...]
    Whh = Whh_ref[...]
    bih = bih_ref[...]
    bhh = bhh_ref[...]
    sup_b = jnp.broadcast_to(support_g, (B, DM))

    h_r = jnp.zeros((B, 2 * DM), jnp.float32)
    c = jnp.zeros((B, 2 * DM), jnp.float32)
    h = query_g
    for _ in range(2):
        gates = (jnp.dot(query_g, Wih.T, preferred_element_type=jnp.float32)
                 + bih
                 + jnp.dot(h_r, Whh.T, preferred_element_type=jnp.float32)
                 + bhh)
        i_g = gates[:, 0:2 * DM]
        f_g = gates[:, 2 * DM:4 * DM]
        g_g = gates[:, 4 * DM:6 * DM]
        o_g = gates[:, 6 * DM:8 * DM]
        c = jax.nn.sigmoid(f_g) * c + jax.nn.sigmoid(i_g) * jnp.tanh(g_g)
        h_new = jax.nn.sigmoid(o_g) * jnp.tanh(c)
        h = query_g + h_new[:, :DM]
        h_r = jnp.concatenate([h, sup_b], axis=1)

    qf = h / jnp.maximum(jnp.linalg.norm(h, axis=-1, keepdims=True), 1e-12)
    sv = support_g[0]
    sv = sv / jnp.maximum(jnp.linalg.norm(sv), 1e-12)
    out_ref[...] = jnp.dot(qf, sv[:, None],
                           preferred_element_type=jnp.float32)[:, 0]


def kernel(query, support, q_l_conn, q_l_deg, q_r_conn, q_r_deg,
           s_l_conn, s_l_deg, s_r_conn, s_r_deg, table,
           gcn_wW, gcn_wb, gcn_b, p1W, p1b, p2W, p2b, ln_g, ln_b,
           Wih, Whh, bih, bhh):
    B = query.shape[0]
    FEW = support.shape[0]

    ids = jnp.zeros((ROWS,), jnp.int32)
    ids = ids.at[0:B].set(query[:, 0].astype(jnp.int32))
    ids = ids.at[B:2 * B].set(query[:, 1].astype(jnp.int32))
    ids = ids.at[2 * B:2 * B + FEW].set(support[:, 0].astype(jnp.int32))
    ids = ids.at[2 * B + 8:2 * B + 8 + FEW].set(support[:, 1].astype(jnp.int32))
    conn = jnp.zeros((ROWS, NB, 2), jnp.int32)
    conn = conn.at[0:B].set(q_l_conn.astype(jnp.int32))
    conn = conn.at[B:2 * B].set(q_r_conn.astype(jnp.int32))
    conn = conn.at[2 * B:2 * B + FEW].set(s_l_conn.astype(jnp.int32))
    conn = conn.at[2 * B + 8:2 * B + 8 + FEW].set(s_r_conn.astype(jnp.int32))

    idx1 = jnp.concatenate(
        [ids[:, None], conn[:, :, 1],
         jnp.zeros((ROWS, W1 - 1 - NB), jnp.int32)], axis=1)
    relids = jnp.concatenate(
        [conn[:, :, 0], jnp.zeros((ROWS, 64 - NB), jnp.int32)], axis=1)

    mean = _sc_encode(table, idx1, relids)

    scores = pl.pallas_call(
        functools.partial(_head_kernel, B=B),
        out_shape=jax.ShapeDtypeStruct((B,), jnp.float32),
        compiler_params=pltpu.CompilerParams(
            vmem_limit_bytes=63 * 1024 * 1024),
    )(mean, gcn_wW, gcn_wb + gcn_b, p1W, p1b, p2W, p2b, ln_g, ln_b,
      Wih, Whh, bih, bhh)
    return scores
